# Initial kernel scaffold; baseline (speedup 1.0000x reference)
import jax
import jax.numpy as jnp
from jax.experimental import pallas as pl


def kernel(input_ids, token_type_ids, word_table, pos_table, type_table, ln_gamma, ln_beta):
    raise NotImplementedError("write your pallas kernel here")


# speedup vs baseline: 6.8734x; 6.8734x over previous
import functools

import jax
import jax.numpy as jnp
from jax import lax
from jax.experimental import pallas as pl
from jax.experimental.pallas import tpu as pltpu
from jax.experimental.pallas import tpu_sc as plsc

HID = 128
SEQ = 200
BATCH = 1024
NTOK = BATCH * SEQ
EPS = 1e-6

NC = 2
NS = 16
NW = NC * NS
TOK_PER_W = NTOK // NW
CHUNK = 128
NSTEP = TOK_PER_W // CHUNK
NSL = HID // 16


def _combine_body(pos_ref, type_ref, out_ref):
    p = pos_ref[0:SEQ, :]
    out_ref[0:SEQ, :] = p + type_ref[0:1, :]
    out_ref[SEQ:2 * SEQ, :] = p + type_ref[1:2, :]


_combine = pl.pallas_call(
    _combine_body,
    out_shape=jax.ShapeDtypeStruct((2 * SEQ, HID), jnp.float32),
)


def _rsqrt16(x):
    i = lax.bitcast_convert_type(x, jnp.int32)
    i = jnp.int32(0x5F3759DF) - lax.shift_right_arithmetic(i, 1)
    y = lax.bitcast_convert_type(i, jnp.float32)
    for _ in range(3):
        y = y * (jnp.float32(1.5) - jnp.float32(0.5) * x * y * y)
    return y


def _sc_body(word_hbm, comb_hbm, ids_hbm, tt_hbm, gamma_hbm, beta_hbm, out_hbm,
             ids_v, tt_v, cidx_v, rows_w, rows_c, out_v, g_v, b_v, sem_w, sem_c):
    wid = lax.axis_index("s") * NC + lax.axis_index("c")
    base = wid * TOK_PER_W

    pltpu.sync_copy(gamma_hbm, g_v)
    pltpu.sync_copy(beta_hbm, b_v)
    g = [g_v[pl.ds(16 * e, 16)] for e in range(NSL)]
    b = [b_v[pl.ds(16 * e, 16)] for e in range(NSL)]
    iot = lax.iota(jnp.int32, 16)
    perms = [lax.bitwise_xor(iot, jnp.int32(k)) for k in (1, 2, 4, 8)]
    inv_h = jnp.float32(1.0 / HID)

    dnums = lax.GatherDimensionNumbers(
        offset_dims=(), collapsed_slice_dims=(0,), start_index_map=(0,))

    def lane_sum(v):
        for p in perms:
            shuf = lax.gather(v, p[:, None], dnums, slice_sizes=(1,),
                              mode=lax.GatherScatterMode.PROMISE_IN_BOUNDS)
            v = v + shuf
        return v

    def step(s, carry):
        gbase = base + s * CHUNK
        pltpu.sync_copy(ids_hbm.at[pl.ds(gbase, CHUNK)], ids_v)
        pltpu.sync_copy(tt_hbm.at[pl.ds(gbase, CHUNK)], tt_v)
        for j in range(CHUNK // 16):
            n = gbase + j * 16 + iot
            pidx = lax.rem(n, SEQ)
            cidx_v[pl.ds(j * 16, 16)] = tt_v[pl.ds(j * 16, 16)] * SEQ + pidx
        cw = pltpu.async_copy(word_hbm.at[ids_v], rows_w, sem_w)
        cc = pltpu.async_copy(comb_hbm.at[cidx_v], rows_c, sem_c)
        cw.wait()
        cc.wait()

        def row(r, rc):
            w = [rows_w[r, pl.ds(16 * e, 16)] + rows_c[r, pl.ds(16 * e, 16)]
                 for e in range(NSL)]
            s0 = (w[0] + w[1]) + (w[2] + w[3])
            s1 = (w[4] + w[5]) + (w[6] + w[7])
            q0 = (w[0] * w[0] + w[1] * w[1]) + (w[2] * w[2] + w[3] * w[3])
            q1 = (w[4] * w[4] + w[5] * w[5]) + (w[6] * w[6] + w[7] * w[7])
            mv = lane_sum(s0 + s1) * inv_h
            var = lane_sum(q0 + q1) * inv_h - mv * mv
            rv = _rsqrt16(var + jnp.float32(EPS))
            for e in range(NSL):
                out_v[r, pl.ds(16 * e, 16)] = (w[e] - mv) * rv * g[e] + b[e]
            return rc

        lax.fori_loop(0, CHUNK, row, 0)
        pltpu.sync_copy(out_v, out_hbm.at[pl.ds(gbase, CHUNK)])
        return carry

    lax.fori_loop(0, NSTEP, step, 0)


_sc_call = pl.kernel(
    _sc_body,
    out_type=jax.ShapeDtypeStruct((NTOK, HID), jnp.float32),
    mesh=plsc.VectorSubcoreMesh(
        core_axis_name="c", subcore_axis_name="s", num_cores=NC,
        num_subcores=NS),
    scratch_types=[
        pltpu.VMEM((CHUNK,), jnp.int32),
        pltpu.VMEM((CHUNK,), jnp.int32),
        pltpu.VMEM((CHUNK,), jnp.int32),
        pltpu.VMEM((CHUNK, HID), jnp.float32),
        pltpu.VMEM((CHUNK, HID), jnp.float32),
        pltpu.VMEM((CHUNK, HID), jnp.float32),
        pltpu.VMEM((HID,), jnp.float32),
        pltpu.VMEM((HID,), jnp.float32),
        pltpu.SemaphoreType.DMA,
        pltpu.SemaphoreType.DMA,
    ],
)


def kernel(input_ids, token_type_ids, word_table, pos_table, type_table,
           ln_gamma, ln_beta):
    comb = _combine(pos_table, type_table)
    ids = input_ids.reshape(-1)
    tt = token_type_ids.reshape(-1)
    out = _sc_call(word_table, comb, ids, tt, ln_gamma, ln_beta)
    return out.reshape(input_ids.shape[0], input_ids.shape[1], HID)


# speedup vs baseline: 9.0960x; 1.3234x over previous
import functools

import jax
import jax.numpy as jnp
from jax import lax
from jax.experimental import pallas as pl
from jax.experimental.pallas import tpu as pltpu
from jax.experimental.pallas import tpu_sc as plsc

HID = 128
SEQ = 200
BATCH = 1024
NTOK = BATCH * SEQ
EPS = 1e-6

NC = 2
NS = 16
NW = NC * NS
TOK_PER_W = NTOK // NW
CHUNK = 128
NSTEP = TOK_PER_W // CHUNK
NSL = HID // 16


def _combine_body(pos_ref, type_ref, out_ref):
    p = pos_ref[0:SEQ, :]
    out_ref[0:SEQ, :] = p + type_ref[0:1, :]
    out_ref[SEQ:2 * SEQ, :] = p + type_ref[1:2, :]


_combine = pl.pallas_call(
    _combine_body,
    out_shape=jax.ShapeDtypeStruct((2 * SEQ, HID), jnp.float32),
)


def _rsqrt16(x):
    i = lax.bitcast_convert_type(x, jnp.int32)
    i = jnp.int32(0x5F3759DF) - lax.shift_right_arithmetic(i, 1)
    y = lax.bitcast_convert_type(i, jnp.float32)
    for _ in range(3):
        y = y * (jnp.float32(1.5) - jnp.float32(0.5) * x * y * y)
    return y


def _sc_body(word_hbm, comb_hbm, ids_hbm, tt_hbm, gamma_hbm, beta_hbm, out_hbm,
             ids_v, cidx_v, rows_w0, rows_w1, rows_c0, rows_c1, out0, out1,
             g_v, b_v, sw0, sw1, sc0, sc1, so0, so1):
    rows_w = (rows_w0, rows_w1)
    rows_c = (rows_c0, rows_c1)
    out_v = (out0, out1)
    sem_w = (sw0, sw1)
    sem_c = (sc0, sc1)
    sem_o = (so0, so1)

    wid = lax.axis_index("s") * NC + lax.axis_index("c")
    base = wid * TOK_PER_W

    pltpu.sync_copy(gamma_hbm, g_v)
    pltpu.sync_copy(beta_hbm, b_v)
    pltpu.sync_copy(ids_hbm.at[wid], ids_v)
    pltpu.sync_copy(tt_hbm.at[wid], cidx_v)

    g = [g_v[pl.ds(16 * e, 16)] for e in range(NSL)]
    b = [b_v[pl.ds(16 * e, 16)] for e in range(NSL)]
    iot = lax.iota(jnp.int32, 16)
    perms = [lax.bitwise_xor(iot, jnp.int32(k)) for k in (1, 2, 4, 8)]
    inv_h = jnp.float32(1.0 / HID)

    dnums = lax.GatherDimensionNumbers(
        offset_dims=(), collapsed_slice_dims=(0,), start_index_map=(0,))

    def lane_sum(v):
        for p in perms:
            shuf = lax.gather(v, p[:, None], dnums, slice_sizes=(1,),
                              mode=lax.GatherScatterMode.PROMISE_IN_BOUNDS)
            v = v + shuf
        return v

    def mkidx(s, carry):
        for j in range(CHUNK // 16):
            n = base + s * CHUNK + j * 16 + iot
            pidx = lax.rem(n, SEQ)
            sl = pl.ds(16 * j, 16)
            cidx_v[s, sl] = cidx_v[s, sl] * SEQ + pidx
        return carry

    lax.fori_loop(0, NSTEP, mkidx, 0)

    def fetch(s, bi):
        pltpu.async_copy(word_hbm.at[ids_v.at[s]], rows_w[bi], sem_w[bi])
        pltpu.async_copy(comb_hbm.at[cidx_v.at[s]], rows_c[bi], sem_c[bi])

    fetch(0, 0)
    fetch(1, 1)

    def body(i, carry):
        for bi in (0, 1):
            s = 2 * i + bi

            @pl.when(s >= 2)
            def _():
                pltpu.make_async_copy(
                    out_v[bi], out_hbm.at[pl.ds(0, CHUNK)], sem_o[bi]).wait()

            pltpu.make_async_copy(
                word_hbm.at[pl.ds(0, CHUNK)], rows_w[bi], sem_w[bi]).wait()
            pltpu.make_async_copy(
                comb_hbm.at[pl.ds(0, CHUNK)], rows_c[bi], sem_c[bi]).wait()

            rw, rc, ov = rows_w[bi], rows_c[bi], out_v[bi]

            def row_pair(k, rcarry):
                for u in (0, 1):
                    r = 2 * k + u
                    w = [rw[r, pl.ds(16 * e, 16)] + rc[r, pl.ds(16 * e, 16)]
                         for e in range(NSL)]
                    s0 = (w[0] + w[1]) + (w[2] + w[3])
                    s1 = (w[4] + w[5]) + (w[6] + w[7])
                    q0 = (w[0] * w[0] + w[1] * w[1]) + (w[2] * w[2] + w[3] * w[3])
                    q1 = (w[4] * w[4] + w[5] * w[5]) + (w[6] * w[6] + w[7] * w[7])
                    mv = lane_sum(s0 + s1) * inv_h
                    var = lane_sum(q0 + q1) * inv_h - mv * mv
                    rv = _rsqrt16(var + jnp.float32(EPS))
                    for e in range(NSL):
                        ov[r, pl.ds(16 * e, 16)] = (w[e] - mv) * rv * g[e] + b[e]
                return rcarry

            lax.fori_loop(0, CHUNK // 2, row_pair, 0)

            gbase = base + s * CHUNK
            pltpu.async_copy(ov, out_hbm.at[pl.ds(gbase, CHUNK)], sem_o[bi])

            @pl.when(s + 2 < NSTEP)
            def _():
                fetch(s + 2, bi)
        return carry

    lax.fori_loop(0, NSTEP // 2, body, 0)
    pltpu.make_async_copy(out_v[0], out_hbm.at[pl.ds(0, CHUNK)], sem_o[0]).wait()
    pltpu.make_async_copy(out_v[1], out_hbm.at[pl.ds(0, CHUNK)], sem_o[1]).wait()


_sc_call = pl.kernel(
    _sc_body,
    out_type=jax.ShapeDtypeStruct((NTOK, HID), jnp.float32),
    mesh=plsc.VectorSubcoreMesh(
        core_axis_name="c", subcore_axis_name="s", num_cores=NC,
        num_subcores=NS),
    scratch_types=[
        pltpu.VMEM((NSTEP, CHUNK), jnp.int32),
        pltpu.VMEM((NSTEP, CHUNK), jnp.int32),
        pltpu.VMEM((CHUNK, HID), jnp.float32),
        pltpu.VMEM((CHUNK, HID), jnp.float32),
        pltpu.VMEM((CHUNK, HID), jnp.float32),
        pltpu.VMEM((CHUNK, HID), jnp.float32),
        pltpu.VMEM((CHUNK, HID), jnp.float32),
        pltpu.VMEM((CHUNK, HID), jnp.float32),
        pltpu.VMEM((HID,), jnp.float32),
        pltpu.VMEM((HID,), jnp.float32),
        pltpu.SemaphoreType.DMA,
        pltpu.SemaphoreType.DMA,
        pltpu.SemaphoreType.DMA,
        pltpu.SemaphoreType.DMA,
        pltpu.SemaphoreType.DMA,
        pltpu.SemaphoreType.DMA,
    ],
)


def kernel(input_ids, token_type_ids, word_table, pos_table, type_table,
           ln_gamma, ln_beta):
    comb = _combine(pos_table, type_table)
    ids = input_ids.reshape(NW, NSTEP, CHUNK)
    tt = token_type_ids.reshape(NW, NSTEP, CHUNK)
    out = _sc_call(word_table, comb, ids, tt, ln_gamma, ln_beta)
    return out.reshape(input_ids.shape[0], input_ids.shape[1], HID)
